# baseline (speedup 1.0000x reference)
import functools

import jax
import jax.numpy as jnp
from jax import lax
from jax.experimental import pallas as pl
from jax.experimental.pallas import tpu as pltpu
from jax.experimental.pallas import tpu_sc as plsc

K_NEIGHBORS = 300
EPS = 1e-6
BLK_B = 128


def _sc_gather(table, idx):
    B = idx.shape[0]
    D = table.shape[1]
    info = plsc.get_sparse_core_info()
    nw = info.num_cores * info.num_subcores
    bw = B // nw
    mesh = plsc.VectorSubcoreMesh(core_axis_name="c", subcore_axis_name="s")

    @functools.partial(
        pl.kernel,
        mesh=mesh,
        out_type=jax.ShapeDtypeStruct((B, D), table.dtype),
        scratch_types=[
            pltpu.VMEM((bw,), jnp.int32),
            pltpu.VMEM((bw, D), table.dtype),
            pltpu.SemaphoreType.DMA,
        ],
    )
    def k(table_hbm, idx_hbm, out_hbm, idx_v, rows_v, sem):
        wid = lax.axis_index("s") * info.num_cores + lax.axis_index("c")
        base = wid * bw
        pltpu.sync_copy(idx_hbm.at[pl.ds(base, bw)], idx_v)
        pltpu.async_copy(table_hbm.at[idx_v], rows_v, sem).wait()
        pltpu.sync_copy(rows_v, out_hbm.at[pl.ds(base, bw)])

    return k(table, idx)


def _tc_body(q_ref, ub_ref, up_ref, o_ref, ubb_ref, nu_ref):
    f32 = jnp.float32
    bf16 = jnp.bfloat16
    U = ub_ref.shape[0]

    @pl.when(pl.program_id(0) == 0)
    def _():
        ubb_ref[...] = ub_ref[...].astype(bf16)
        ones8 = jnp.ones((8, ub_ref.shape[1]), bf16)
        nu_ref[...] = lax.dot_general(ones8, ubb_ref[...],
                                      (((1,), (1,)), ((), ())),
                                      preferred_element_type=f32)

    qb = q_ref[...]
    ubb = ubb_ref[...]
    qbb = qb.astype(bf16)
    dots = lax.dot_general(qbb, ubb, (((1,), (1,)), ((), ())),
                           preferred_element_type=f32)
    nu = nu_ref[0:1]
    nq = jnp.sum(qb, axis=1, keepdims=True)
    denom = jnp.sqrt(nq) * jnp.sqrt(nu) + EPS
    sim = dots / denom
    si = lax.bitcast_convert_type(sim, jnp.int32)
    i16 = jnp.int16

    hi = lax.shift_right_logical(si, 16).astype(i16)
    lo = (si ^ 0x8000).astype(i16)

    def _tree_count(m16):
        x = m16
        w = x.shape[1]
        while w > 16:
            w //= 2
            x = x[:, :w] + x[:, w:]
        return jnp.sum(x.astype(jnp.int32), axis=1, keepdims=True)

    zero_col = jnp.zeros((qb.shape[0], 1), jnp.int32)

    def p1_body(i, cur):
        cand = cur | lax.shift_right_logical(jnp.int32(2 ** 13), i)
        m = (hi >= cand.astype(i16)).astype(i16)
        return jnp.where(_tree_count(m) >= K_NEIGHBORS, cand, cur)

    th = lax.fori_loop(0, 14, p1_body, zero_col)
    th16 = th.astype(i16)
    n_hi_gt = _tree_count((hi > th16).astype(i16))
    eqm = (hi == th16).astype(i16)

    def p2_body(i, cur):
        cand = cur | lax.shift_right_logical(jnp.int32(2 ** 15), i)
        cs = (cand ^ 0x8000).astype(i16)
        m = jnp.where(lo >= cs, eqm, jnp.int16(0))
        return jnp.where(n_hi_gt + _tree_count(m) >= K_NEIGHBORS, cand, cur)

    tl = lax.fori_loop(0, 16, p2_body, zero_col)
    tl16 = (tl ^ 0x8000).astype(i16)

    hi_eq = hi == th16
    gt = (hi > th16) | (hi_eq & (lo > tl16))
    eq = hi_eq & (lo == tl16)
    ng = _tree_count(gt.astype(i16))
    r = eq.astype(i16)
    lane = lax.broadcasted_iota(i16, r.shape, 1)
    s = 1
    while s < r.shape[1]:
        r = r + jnp.where(lane >= jnp.int16(s), pltpu.roll(r, s, axis=1),
                          jnp.int16(0))
        s *= 2
    need = (K_NEIGHBORS - ng).astype(i16)
    sel = jnp.logical_or(gt, jnp.logical_and(eq, r <= need))
    w = jnp.where(sel, sim, 0.0)
    o_ref[...] = lax.dot_general(w, up_ref[...], (((1,), (0,)), ((), ())),
                                 preferred_element_type=f32)


def _tc_main(q, ub_p, up):
    B = q.shape[0]
    U, IP = ub_p.shape
    I = up.shape[1]
    return pl.pallas_call(
        _tc_body,
        grid=(B // BLK_B,),
        in_specs=[
            pl.BlockSpec((BLK_B, IP), lambda i: (i, 0)),
            pl.BlockSpec((U, IP), lambda i: (0, 0)),
            pl.BlockSpec((U, I), lambda i: (0, 0)),
        ],
        out_specs=pl.BlockSpec((BLK_B, I), lambda i: (i, 0)),
        out_shape=jax.ShapeDtypeStruct((B, I), jnp.float32),
        scratch_shapes=[
            pltpu.VMEM((U, IP), jnp.bfloat16),
            pltpu.VMEM((8, U), jnp.float32),
        ],
    )(q, ub_p, up)


def kernel(user_bin, user_pref, user_id):
    U, I = user_bin.shape
    IP = 1024
    ub_p = jnp.pad(user_bin, ((0, 0), (0, IP - I)))
    q = _sc_gather(ub_p, user_id.astype(jnp.int32))
    return _tc_main(q, ub_p, user_pref)


# speedup vs baseline: 1.1637x; 1.1637x over previous
import functools

import jax
import jax.numpy as jnp
from jax import lax
from jax.experimental import pallas as pl
from jax.experimental.pallas import tpu as pltpu
from jax.experimental.pallas import tpu_sc as plsc

K_NEIGHBORS = 300
EPS = 1e-6
BLK_B = 128


def _sc_gather(table, idx):
    B = idx.shape[0]
    D = table.shape[1]
    info = plsc.get_sparse_core_info()
    nw = info.num_cores * info.num_subcores
    bw = B // nw
    mesh = plsc.VectorSubcoreMesh(core_axis_name="c", subcore_axis_name="s")

    @functools.partial(
        pl.kernel,
        mesh=mesh,
        out_type=jax.ShapeDtypeStruct((B, D), table.dtype),
        scratch_types=[
            pltpu.VMEM((bw,), jnp.int32),
            pltpu.VMEM((bw, D), table.dtype),
            pltpu.SemaphoreType.DMA,
        ],
    )
    def k(table_hbm, idx_hbm, out_hbm, idx_v, rows_v, sem):
        wid = lax.axis_index("s") * info.num_cores + lax.axis_index("c")
        base = wid * bw
        pltpu.sync_copy(idx_hbm.at[pl.ds(base, bw)], idx_v)
        pltpu.async_copy(table_hbm.at[idx_v], rows_v, sem).wait()
        pltpu.sync_copy(rows_v, out_hbm.at[pl.ds(base, bw)])

    return k(table, idx)


def _tc_body(q_ref, ub_ref, up_ref, o_ref, ubb_ref, nu_ref):
    f32 = jnp.float32
    bf16 = jnp.bfloat16
    U = ub_ref.shape[0]

    @pl.when(pl.program_id(0) == 0)
    def _():
        ubb_ref[...] = ub_ref[...].astype(bf16)
        ones8 = jnp.ones((8, ub_ref.shape[1]), bf16)
        nu_ref[...] = lax.dot_general(ones8, ubb_ref[...],
                                      (((1,), (1,)), ((), ())),
                                      preferred_element_type=f32)

    qb = q_ref[...]
    ubb = ubb_ref[...]
    qbb = qb.astype(bf16)
    dots = lax.dot_general(qbb, ubb, (((1,), (1,)), ((), ())),
                           preferred_element_type=f32)
    nu = nu_ref[0:1]
    nq = jnp.sum(qb, axis=1, keepdims=True)
    denom = jnp.sqrt(nq) * jnp.sqrt(nu) + EPS
    sim = dots / denom
    si = lax.bitcast_convert_type(sim, jnp.int32)
    i16 = jnp.int16

    def _count_ge(thresh):
        return jnp.sum(jnp.where(si >= thresh, 1.0, 0.0),
                       axis=1, keepdims=True)

    def bs_body(i, cur):
        cand = cur | lax.shift_right_logical(jnp.int32(2 ** 30), i)
        return jnp.where(_count_ge(cand) >= K_NEIGHBORS, cand, cur)

    t = lax.fori_loop(1, 31, bs_body,
                      jnp.zeros((qb.shape[0], 1), jnp.int32))
    gt = si > t
    eq = si == t
    ng = _count_ge(t + 1)
    r = eq.astype(i16)
    lane = lax.broadcasted_iota(i16, r.shape, 1)
    s = 1
    while s < r.shape[1]:
        r = r + jnp.where(lane >= jnp.int16(s), pltpu.roll(r, s, axis=1),
                          jnp.int16(0))
        s *= 2
    need = (K_NEIGHBORS - ng).astype(i16)
    sel = jnp.logical_or(gt, jnp.logical_and(eq, r <= need))
    w = jnp.where(sel, sim, 0.0)
    o_ref[...] = lax.dot_general(w, up_ref[...], (((1,), (0,)), ((), ())),
                                 preferred_element_type=f32)


def _tc_main(q, ub_p, up):
    B = q.shape[0]
    U, IP = ub_p.shape
    I = up.shape[1]
    return pl.pallas_call(
        _tc_body,
        grid=(B // BLK_B,),
        in_specs=[
            pl.BlockSpec((BLK_B, IP), lambda i: (i, 0)),
            pl.BlockSpec((U, IP), lambda i: (0, 0)),
            pl.BlockSpec((U, I), lambda i: (0, 0)),
        ],
        out_specs=pl.BlockSpec((BLK_B, I), lambda i: (i, 0)),
        out_shape=jax.ShapeDtypeStruct((B, I), jnp.float32),
        scratch_shapes=[
            pltpu.VMEM((U, IP), jnp.bfloat16),
            pltpu.VMEM((8, U), jnp.float32),
        ],
    )(q, ub_p, up)


def kernel(user_bin, user_pref, user_id):
    U, I = user_bin.shape
    IP = 1024
    ub_p = jnp.pad(user_bin, ((0, 0), (0, IP - I)))
    q = _sc_gather(ub_p, user_id.astype(jnp.int32))
    return _tc_main(q, ub_p, user_pref)


# speedup vs baseline: 1.3077x; 1.1237x over previous
import functools

import jax
import jax.numpy as jnp
from jax import lax
from jax.experimental import pallas as pl
from jax.experimental.pallas import tpu as pltpu
from jax.experimental.pallas import tpu_sc as plsc

K_NEIGHBORS = 300
EPS = 1e-6
BLK_B = 128


def _sc_gather(table, idx):
    B = idx.shape[0]
    D = table.shape[1]
    info = plsc.get_sparse_core_info()
    nw = info.num_cores * info.num_subcores
    bw = B // nw
    mesh = plsc.VectorSubcoreMesh(core_axis_name="c", subcore_axis_name="s")

    @functools.partial(
        pl.kernel,
        mesh=mesh,
        out_type=jax.ShapeDtypeStruct((B, D), table.dtype),
        scratch_types=[
            pltpu.VMEM((bw,), jnp.int32),
            pltpu.VMEM((bw, D), table.dtype),
            pltpu.SemaphoreType.DMA,
        ],
    )
    def k(table_hbm, idx_hbm, out_hbm, idx_v, rows_v, sem):
        wid = lax.axis_index("s") * info.num_cores + lax.axis_index("c")
        base = wid * bw
        pltpu.sync_copy(idx_hbm.at[pl.ds(base, bw)], idx_v)
        pltpu.async_copy(table_hbm.at[idx_v], rows_v, sem).wait()
        pltpu.sync_copy(rows_v, out_hbm.at[pl.ds(base, bw)])

    return k(table, idx)


def _tc_body(q_ref, ub_ref, up_ref, o_ref, ubb_ref, nu_ref):
    f32 = jnp.float32
    bf16 = jnp.bfloat16
    U = ub_ref.shape[0]

    @pl.when(pl.program_id(0) == 0)
    def _():
        ubb_ref[...] = ub_ref[...].astype(bf16)
        ones8 = jnp.ones((8, ub_ref.shape[1]), bf16)
        nu_ref[...] = lax.dot_general(ones8, ubb_ref[...],
                                      (((1,), (1,)), ((), ())),
                                      preferred_element_type=f32)

    qb = q_ref[...]
    ubb = ubb_ref[...]
    qbb = qb.astype(bf16)
    dots = lax.dot_general(qbb, ubb, (((1,), (1,)), ((), ())),
                           preferred_element_type=f32)
    nu = nu_ref[0:1]
    nq = jnp.sum(qb, axis=1, keepdims=True)
    denom = jnp.sqrt(nq) * jnp.sqrt(nu) + EPS
    sim = dots / denom
    si = lax.bitcast_convert_type(sim, jnp.int32)
    i16 = jnp.int16

    def _count_ge(thresh):
        return jnp.sum(jnp.where(si >= thresh, 1.0, 0.0),
                       axis=1, keepdims=True)

    nch = 4
    h = qb.shape[0] // nch
    si_ch = [si[j * h:(j + 1) * h] for j in range(nch)]
    cur = [jnp.zeros((h, 1), jnp.int32) for _ in range(nch)]
    for i in range(1, 31):
        bit = jnp.int32(2 ** 30 >> i)
        cands = [c | bit for c in cur]
        cnts = [jnp.sum(jnp.where(s >= cd, 1.0, 0.0), axis=1, keepdims=True)
                for s, cd in zip(si_ch, cands)]
        cur = [jnp.where(cn >= K_NEIGHBORS, cd, c)
               for cn, cd, c in zip(cnts, cands, cur)]
    t = jnp.concatenate(cur, axis=0)
    gt = si > t
    eq = si == t
    ng = _count_ge(t + 1)
    r = eq.astype(i16)
    lane = lax.broadcasted_iota(i16, r.shape, 1)
    s = 1
    while s < r.shape[1]:
        r = r + jnp.where(lane >= jnp.int16(s), pltpu.roll(r, s, axis=1),
                          jnp.int16(0))
        s *= 2
    need = (K_NEIGHBORS - ng).astype(i16)
    sel = jnp.logical_or(gt, jnp.logical_and(eq, r <= need))
    w = jnp.where(sel, sim, 0.0)
    o_ref[...] = lax.dot_general(w, up_ref[...], (((1,), (0,)), ((), ())),
                                 preferred_element_type=f32)


def _tc_main(q, ub_p, up):
    B = q.shape[0]
    U, IP = ub_p.shape
    I = up.shape[1]
    return pl.pallas_call(
        _tc_body,
        grid=(B // BLK_B,),
        in_specs=[
            pl.BlockSpec((BLK_B, IP), lambda i: (i, 0)),
            pl.BlockSpec((U, IP), lambda i: (0, 0)),
            pl.BlockSpec((U, I), lambda i: (0, 0)),
        ],
        out_specs=pl.BlockSpec((BLK_B, I), lambda i: (i, 0)),
        out_shape=jax.ShapeDtypeStruct((B, I), jnp.float32),
        scratch_shapes=[
            pltpu.VMEM((U, IP), jnp.bfloat16),
            pltpu.VMEM((8, U), jnp.float32),
        ],
    )(q, ub_p, up)


def kernel(user_bin, user_pref, user_id):
    U, I = user_bin.shape
    IP = 1024
    ub_p = jnp.pad(user_bin, ((0, 0), (0, IP - I)))
    q = _sc_gather(ub_p, user_id.astype(jnp.int32))
    return _tc_main(q, ub_p, user_pref)


# speedup vs baseline: 1.4249x; 1.0896x over previous
import functools

import jax
import jax.numpy as jnp
from jax import lax
from jax.experimental import pallas as pl
from jax.experimental.pallas import tpu as pltpu
from jax.experimental.pallas import tpu_sc as plsc

K_NEIGHBORS = 300
EPS = 1e-6
BLK_B = 128


def _sc_gather(table, idx):
    B = idx.shape[0]
    D = table.shape[1]
    info = plsc.get_sparse_core_info()
    nw = info.num_cores * info.num_subcores
    bw = B // nw
    mesh = plsc.VectorSubcoreMesh(core_axis_name="c", subcore_axis_name="s")

    @functools.partial(
        pl.kernel,
        mesh=mesh,
        out_type=jax.ShapeDtypeStruct((B, D), table.dtype),
        scratch_types=[
            pltpu.VMEM((bw,), jnp.int32),
            pltpu.VMEM((bw, D), table.dtype),
            pltpu.SemaphoreType.DMA,
        ],
    )
    def k(table_hbm, idx_hbm, out_hbm, idx_v, rows_v, sem):
        wid = lax.axis_index("s") * info.num_cores + lax.axis_index("c")
        base = wid * bw
        pltpu.sync_copy(idx_hbm.at[pl.ds(base, bw)], idx_v)
        pltpu.async_copy(table_hbm.at[idx_v], rows_v, sem).wait()
        pltpu.sync_copy(rows_v, out_hbm.at[pl.ds(base, bw)])

    return k(table, idx)


def _tc_body(q_ref, ub_ref, up_ref, o_ref, ubb_ref, nu_ref):
    f32 = jnp.float32
    bf16 = jnp.bfloat16
    U = ub_ref.shape[0]

    @pl.when(pl.program_id(0) == 0)
    def _():
        ubb_ref[...] = ub_ref[...].astype(bf16)
        ones8 = jnp.ones((8, ub_ref.shape[1]), bf16)
        nu_ref[...] = lax.dot_general(ones8, ubb_ref[...],
                                      (((1,), (1,)), ((), ())),
                                      preferred_element_type=f32)

    qb = q_ref[...]
    ubb = ubb_ref[...]
    qbb = qb.astype(bf16)
    dots = lax.dot_general(qbb, ubb, (((1,), (1,)), ((), ())),
                           preferred_element_type=f32)
    nu = nu_ref[0:1]
    nq = jnp.sum(qb, axis=1, keepdims=True)
    denom = jnp.sqrt(nq) * jnp.sqrt(nu) + EPS
    sim = dots / denom
    si = lax.bitcast_convert_type(sim, jnp.int32)
    i16 = jnp.int16

    nch = 2
    h = qb.shape[0] // nch
    si_ch = [si[j * h:(j + 1) * h] for j in range(nch)]
    his = [lax.shift_right_logical(s, 16).astype(i16) for s in si_ch]
    los = [(s ^ 0x8000).astype(i16) for s in si_ch]

    def _tree16(m):
        x = m
        w = x.shape[1]
        while w > 128:
            w //= 2
            x = x[:, :w] + x[:, w:]
        return jnp.sum(x.astype(jnp.int32), axis=1, keepdims=True)

    cur = [jnp.zeros((h, 1), jnp.int32) for _ in range(nch)]
    for i in range(14):
        bit = jnp.int32(0x2000 >> i)
        cands = [c | bit for c in cur]
        cnts = [_tree16((hh >= cd.astype(i16)).astype(i16))
                for hh, cd in zip(his, cands)]
        cur = [jnp.where(cn >= K_NEIGHBORS, cd, c)
               for cn, cd, c in zip(cnts, cands, cur)]
    ths = cur
    th16s = [c.astype(i16) for c in ths]
    g1s = [_tree16((hh > t16).astype(i16)) for hh, t16 in zip(his, th16s)]
    eqms = [(hh == t16).astype(i16) for hh, t16 in zip(his, th16s)]

    cur = [jnp.zeros((h, 1), jnp.int32) for _ in range(nch)]
    for i in range(16):
        bit = jnp.int32(0x8000 >> i)
        cands = [c | bit for c in cur]
        cnts = [_tree16(jnp.where(ll >= (cd ^ 0x8000).astype(i16),
                                  em, jnp.int16(0)))
                for ll, em, cd in zip(los, eqms, cands)]
        cur = [jnp.where(g + cn >= K_NEIGHBORS, cd, c)
               for g, cn, cd, c in zip(g1s, cnts, cands, cur)]
    tl16s = [(c ^ 0x8000).astype(i16) for c in cur]

    gt = jnp.concatenate(
        [(hh > t16) | ((hh == t16) & (ll > l16))
         for hh, ll, t16, l16 in zip(his, los, th16s, tl16s)], axis=0)
    eq = jnp.concatenate(
        [(hh == t16) & (ll == l16)
         for hh, ll, t16, l16 in zip(his, los, th16s, tl16s)], axis=0)
    ng = jnp.concatenate(
        [g + _tree16(((hh == t16) & (ll > l16)).astype(i16))
         for g, hh, ll, t16, l16 in zip(g1s, his, los, th16s, tl16s)],
        axis=0)
    r = eq.astype(i16)
    lane = lax.broadcasted_iota(i16, r.shape, 1)
    s = 1
    while s < r.shape[1]:
        r = r + jnp.where(lane >= jnp.int16(s), pltpu.roll(r, s, axis=1),
                          jnp.int16(0))
        s *= 2
    need = (K_NEIGHBORS - ng).astype(i16)
    sel = jnp.logical_or(gt, jnp.logical_and(eq, r <= need))
    w = jnp.where(sel, sim, 0.0)
    o_ref[...] = lax.dot_general(w, up_ref[...], (((1,), (0,)), ((), ())),
                                 preferred_element_type=f32)


def _tc_main(q, ub_p, up):
    B = q.shape[0]
    U, IP = ub_p.shape
    I = up.shape[1]
    return pl.pallas_call(
        _tc_body,
        grid=(B // BLK_B,),
        in_specs=[
            pl.BlockSpec((BLK_B, IP), lambda i: (i, 0)),
            pl.BlockSpec((U, IP), lambda i: (0, 0)),
            pl.BlockSpec((U, I), lambda i: (0, 0)),
        ],
        out_specs=pl.BlockSpec((BLK_B, I), lambda i: (i, 0)),
        out_shape=jax.ShapeDtypeStruct((B, I), jnp.float32),
        scratch_shapes=[
            pltpu.VMEM((U, IP), jnp.bfloat16),
            pltpu.VMEM((8, U), jnp.float32),
        ],
    )(q, ub_p, up)


def kernel(user_bin, user_pref, user_id):
    U, I = user_bin.shape
    IP = 1024
    ub_p = jnp.pad(user_bin, ((0, 0), (0, IP - I)))
    q = _sc_gather(ub_p, user_id.astype(jnp.int32))
    return _tc_main(q, ub_p, user_pref)


# speedup vs baseline: 1.4716x; 1.0328x over previous
import functools

import jax
import jax.numpy as jnp
from jax import lax
from jax.experimental import pallas as pl
from jax.experimental.pallas import tpu as pltpu
from jax.experimental.pallas import tpu_sc as plsc

K_NEIGHBORS = 300
EPS = 1e-6
BLK_B = 128


def _sc_gather(table, idx):
    B = idx.shape[0]
    D = table.shape[1]
    info = plsc.get_sparse_core_info()
    nw = info.num_cores * info.num_subcores
    bw = B // nw
    mesh = plsc.VectorSubcoreMesh(core_axis_name="c", subcore_axis_name="s")

    @functools.partial(
        pl.kernel,
        mesh=mesh,
        out_type=jax.ShapeDtypeStruct((B, D), table.dtype),
        scratch_types=[
            pltpu.VMEM((bw,), jnp.int32),
            pltpu.VMEM((bw, D), table.dtype),
            pltpu.SemaphoreType.DMA,
        ],
    )
    def k(table_hbm, idx_hbm, out_hbm, idx_v, rows_v, sem):
        wid = lax.axis_index("s") * info.num_cores + lax.axis_index("c")
        base = wid * bw
        pltpu.sync_copy(idx_hbm.at[pl.ds(base, bw)], idx_v)
        pltpu.async_copy(table_hbm.at[idx_v], rows_v, sem).wait()
        pltpu.sync_copy(rows_v, out_hbm.at[pl.ds(base, bw)])

    return k(table, idx)


def _tc_body(q_ref, ub_ref, up_ref, o_ref, ubb_ref, nu_ref):
    f32 = jnp.float32
    bf16 = jnp.bfloat16
    U = ub_ref.shape[0]

    @pl.when(pl.program_id(0) == 0)
    def _():
        ubb_ref[...] = ub_ref[...].astype(bf16)
        ones8 = jnp.ones((8, ub_ref.shape[1]), bf16)
        nu_ref[...] = lax.dot_general(ones8, ubb_ref[...],
                                      (((1,), (1,)), ((), ())),
                                      preferred_element_type=f32)

    qb = q_ref[...]
    ubb = ubb_ref[...]
    qbb = qb.astype(bf16)
    dots = lax.dot_general(qbb, ubb, (((1,), (1,)), ((), ())),
                           preferred_element_type=f32)
    nu = nu_ref[0:1]
    nq = jnp.sum(qb, axis=1, keepdims=True)
    denom = jnp.sqrt(nq) * jnp.sqrt(nu) + EPS
    sim = dots / denom
    si = lax.bitcast_convert_type(sim, jnp.int32)
    i16 = jnp.int16

    nch = 2
    h = qb.shape[0] // nch
    si_ch = [si[j * h:(j + 1) * h] for j in range(nch)]
    his = [lax.shift_right_logical(s, 16).astype(i16) for s in si_ch]
    los = [(s ^ 0x8000).astype(i16) for s in si_ch]

    def _tree16(m):
        x = m
        w = x.shape[1]
        while w > 128:
            w //= 2
            x = x[:, :w] + x[:, w:]
        return jnp.sum(x.astype(jnp.int32), axis=1, keepdims=True)

    cur = [jnp.zeros((h, 1), jnp.int32) for _ in range(nch)]
    for i in range(14):
        bit = jnp.int32(0x2000 >> i)
        cands = [c | bit for c in cur]
        cnts = [_tree16((hh >= cd.astype(i16)).astype(i16))
                for hh, cd in zip(his, cands)]
        cur = [jnp.where(cn >= K_NEIGHBORS, cd, c)
               for cn, cd, c in zip(cnts, cands, cur)]
    ths = cur
    th16s = [c.astype(i16) for c in ths]
    g1s = [_tree16((hh > t16).astype(i16)) for hh, t16 in zip(his, th16s)]
    eqms = [(hh == t16).astype(i16) for hh, t16 in zip(his, th16s)]

    cur = [jnp.zeros((h, 1), jnp.int32) for _ in range(nch)]
    for i in range(16):
        bit = jnp.int32(0x8000 >> i)
        cands = [c | bit for c in cur]
        cnts = [_tree16(jnp.where(ll >= (cd ^ 0x8000).astype(i16),
                                  em, jnp.int16(0)))
                for ll, em, cd in zip(los, eqms, cands)]
        cur = [jnp.where(g + cn >= K_NEIGHBORS, cd, c)
               for g, cn, cd, c in zip(g1s, cnts, cands, cur)]
    tl16s = [(c ^ 0x8000).astype(i16) for c in cur]

    gts = [(hh > t16) | ((hh == t16) & (ll > l16))
           for hh, ll, t16, l16 in zip(his, los, th16s, tl16s)]
    eqs = [(hh == t16) & (ll == l16)
           for hh, ll, t16, l16 in zip(his, los, th16s, tl16s)]
    ngs = [g + _tree16(((hh == t16) & (ll > l16)).astype(i16))
           for g, hh, ll, t16, l16 in zip(g1s, his, los, th16s, tl16s)]
    needs = [K_NEIGHBORS - n for n in ngs]

    lane = lax.broadcasted_iota(i16, (h, eqs[0].shape[1]), 1)
    eq16s = [e.astype(i16) for e in eqs]
    cur = [jnp.zeros((h, 1), jnp.int32) for _ in range(nch)]
    for i in range(12):
        low = jnp.int32((0x800 >> i) - 1)
        cands = [c | low for c in cur]
        cnts = [_tree16(jnp.where(lane <= cd.astype(i16), e16, jnp.int16(0)))
                for e16, cd in zip(eq16s, cands)]
        cur = [jnp.where(cn >= nd, c, c | (low + 1))
               for cn, nd, c, in zip(cnts, needs, cur)]
    sel = jnp.concatenate(
        [g | (e & (lane <= j.astype(i16)))
         for g, e, j in zip(gts, eqs, cur)], axis=0)
    w = jnp.where(sel, sim, 0.0)
    o_ref[...] = lax.dot_general(w, up_ref[...], (((1,), (0,)), ((), ())),
                                 preferred_element_type=f32)


def _tc_main(q, ub_p, up):
    B = q.shape[0]
    U, IP = ub_p.shape
    I = up.shape[1]
    return pl.pallas_call(
        _tc_body,
        grid=(B // BLK_B,),
        in_specs=[
            pl.BlockSpec((BLK_B, IP), lambda i: (i, 0)),
            pl.BlockSpec((U, IP), lambda i: (0, 0)),
            pl.BlockSpec((U, I), lambda i: (0, 0)),
        ],
        out_specs=pl.BlockSpec((BLK_B, I), lambda i: (i, 0)),
        out_shape=jax.ShapeDtypeStruct((B, I), jnp.float32),
        scratch_shapes=[
            pltpu.VMEM((U, IP), jnp.bfloat16),
            pltpu.VMEM((8, U), jnp.float32),
        ],
    )(q, ub_p, up)


def kernel(user_bin, user_pref, user_id):
    U, I = user_bin.shape
    IP = 1024
    ub_p = jnp.pad(user_bin, ((0, 0), (0, IP - I)))
    q = _sc_gather(ub_p, user_id.astype(jnp.int32))
    return _tc_main(q, ub_p, user_pref)
